# baseline (speedup 1.0000x reference)
import functools

import jax
import jax.numpy as jnp
from jax import lax
from jax.experimental import pallas as pl
from jax.experimental.pallas import tpu as pltpu
from jax.experimental.pallas import tpu_sc as plsc

DIM = 64
NC, NS = 2, 16
NW = NC * NS
CHUNK = 128


@functools.lru_cache(maxsize=None)
def _make_sc_gather(B: int, V: int):
    assert B % (NW * CHUNK) == 0
    n_chunks = B // (NW * CHUNK)
    b_per_w = n_chunks * CHUNK
    mesh = plsc.VectorSubcoreMesh(core_axis_name="c", subcore_axis_name="s")

    @functools.partial(
        pl.kernel,
        mesh=mesh,
        compiler_params=pltpu.CompilerParams(use_tc_tiling_on_sc=False),
        out_type=jax.ShapeDtypeStruct((B, DIM), jnp.float32),
        scratch_types=[
            pltpu.VMEM((n_chunks, CHUNK), jnp.int32),
            pltpu.VMEM((CHUNK, DIM), jnp.float32),
            pltpu.SemaphoreType.DMA,
        ],
    )
    def k(idx_hbm, table_hbm, out_hbm, idx_v, rows_v, gsem):
        wid = lax.axis_index("s") * NC + lax.axis_index("c")
        pltpu.sync_copy(idx_hbm.at[wid], idx_v)
        base = wid * b_per_w

        def body(c, carry):
            pltpu.async_copy(table_hbm.at[idx_v.at[c]], rows_v, gsem).wait()
            pltpu.sync_copy(rows_v, out_hbm.at[pl.ds(base + c * CHUNK, CHUNK)])
            return carry

        lax.fori_loop(0, n_chunks, body, 0)

    return k


def kernel(inputs, table):
    bt, h = inputs.shape
    b = bt * h
    idx = inputs.reshape(NW, b // (NW * CHUNK), CHUNK)
    out = _make_sc_gather(b, table.shape[0])(idx, table)
    return out.reshape(bt, h, DIM)


# speedup vs baseline: 1.0400x; 1.0400x over previous
import functools

import jax
import jax.numpy as jnp
from jax import lax
from jax.experimental import pallas as pl
from jax.experimental.pallas import tpu as pltpu
from jax.experimental.pallas import tpu_sc as plsc

DIM = 64
NC, NS = 2, 16
NW = NC * NS
NBUF = 8
LOOKAHEAD = 4


@functools.lru_cache(maxsize=None)
def _make_sc_gather(batch: int, hist: int, vocab: int):
    assert batch % NW == 0
    rows_per_w = batch // NW
    mesh = plsc.VectorSubcoreMesh(core_axis_name="c", subcore_axis_name="s")

    @functools.partial(
        pl.kernel,
        mesh=mesh,
        compiler_params=pltpu.CompilerParams(use_tc_tiling_on_sc=False),
        out_type=jax.ShapeDtypeStruct((batch, hist, DIM), jnp.float32),
        scratch_types=[
            pltpu.VMEM((rows_per_w, hist), jnp.int32),
            pltpu.VMEM((NBUF, hist, DIM), jnp.float32),
        ]
        + [pltpu.SemaphoreType.DMA] * NBUF
        + [pltpu.SemaphoreType.DMA] * NBUF,
    )
    def k(idx_hbm, table_hbm, out_hbm, idx_v, rows_v, *sems):
        gsem = sems[:NBUF]
        wsem = sems[NBUF:]
        wid = lax.axis_index("s") * NC + lax.axis_index("c")
        base = wid * rows_per_w
        pltpu.sync_copy(idx_hbm.at[pl.ds(base, rows_per_w)], idx_v)

        def gather(b, i):
            pltpu.async_copy(table_hbm.at[idx_v.at[b]], rows_v.at[i], gsem[i])

        for b in range(LOOKAHEAD):
            gather(b, b)

        def slot(b, i):
            pltpu.make_async_copy(
                table_hbm.at[idx_v.at[b]], rows_v.at[i], gsem[i]
            ).wait()
            pltpu.async_copy(rows_v.at[i], out_hbm.at[base + b], wsem[i])
            g = b + LOOKAHEAD
            j = (i + LOOKAHEAD) % NBUF

            @pl.when(g < rows_per_w)
            def _():
                @pl.when(g >= NBUF)
                def _():
                    pltpu.make_async_copy(
                        rows_v.at[j], out_hbm.at[base + b], wsem[j]
                    ).wait()

                gather(g, j)

        def outer(k8, carry):
            for i in range(NBUF):
                slot(k8 * NBUF + i, i)
            return carry

        lax.fori_loop(0, rows_per_w // NBUF, outer, 0)

        for i in range(NBUF):
            pltpu.make_async_copy(rows_v.at[i], out_hbm.at[base], wsem[i]).wait()

    return k


def kernel(inputs, table):
    batch, hist = inputs.shape
    return _make_sc_gather(batch, hist, table.shape[0])(inputs, table)


# speedup vs baseline: 1.0446x; 1.0044x over previous
import functools

import jax
import jax.numpy as jnp
from jax import lax
from jax.experimental import pallas as pl
from jax.experimental.pallas import tpu as pltpu
from jax.experimental.pallas import tpu_sc as plsc

DIM = 64
NC, NS = 2, 16
NW = NC * NS
NBUF = 8
LOOKAHEAD = 4


@functools.lru_cache(maxsize=None)
def _make_sc_gather(batch: int, hist: int, vocab: int):
    assert batch % NW == 0
    bw = batch // NW
    mesh = plsc.VectorSubcoreMesh(core_axis_name="c", subcore_axis_name="s")

    @functools.partial(
        pl.kernel,
        mesh=mesh,
        compiler_params=pltpu.CompilerParams(use_tc_tiling_on_sc=False),
        out_type=jax.ShapeDtypeStruct((batch, hist, DIM), jnp.float32),
        scratch_types=[
            pltpu.VMEM((hist, bw), jnp.int32),
            pltpu.VMEM((NBUF, bw, DIM), jnp.float32),
        ]
        + [pltpu.SemaphoreType.DMA] * NBUF
        + [pltpu.SemaphoreType.DMA] * NBUF,
    )
    def k(idx_hbm, table_hbm, out_hbm, idx_v, rows_v, *sems):
        gsem = sems[:NBUF]
        wsem = sems[NBUF:]
        wid = lax.axis_index("s") * NC + lax.axis_index("c")
        base = wid * bw
        pltpu.sync_copy(idx_hbm.at[:, pl.ds(base, bw)], idx_v)

        def gather(h, i):
            pltpu.async_copy(table_hbm.at[idx_v.at[h]], rows_v.at[i], gsem[i])

        for h in range(LOOKAHEAD):
            gather(h, h)

        for h in range(hist):
            i = h % NBUF
            pltpu.make_async_copy(
                table_hbm.at[idx_v.at[h]], rows_v.at[i], gsem[i]
            ).wait()
            pltpu.async_copy(rows_v.at[i], out_hbm.at[pl.ds(base, bw), h], wsem[i])
            g = h + LOOKAHEAD
            if g < hist:
                j = g % NBUF
                if g >= NBUF:
                    pltpu.make_async_copy(
                        rows_v.at[j], out_hbm.at[pl.ds(base, bw), 0], wsem[j]
                    ).wait()
                gather(g, j)

        for t in range(min(NBUF, hist)):
            i = (hist - 1 - t) % NBUF
            pltpu.make_async_copy(
                rows_v.at[i], out_hbm.at[pl.ds(base, bw), 0], wsem[i]
            ).wait()

    return k


def kernel(inputs, table):
    batch, hist = inputs.shape
    return _make_sc_gather(batch, hist, table.shape[0])(inputs.T, table)
